# baseline (speedup 1.0000x reference)
import functools
import jax
import jax.numpy as jnp
from jax import lax
from jax.experimental import pallas as pl
from jax.experimental.pallas import tpu as pltpu
from jax.experimental.pallas import tpu_sc as plsc

_B, _S, _D, _L = 128, 512, 1024, 256
_NW = 32
_DT = _D // _NW


def _sc_body(x_hbm, pos_hbm, lvl_hbm, out_hbm, x_v, pos_v, lvl_v, idx_v, out_v):
    wid = lax.axis_index("s") * 2 + lax.axis_index("c")
    pltpu.sync_copy(x_hbm, x_v)
    pltpu.sync_copy(pos_hbm.at[wid], pos_v)
    pltpu.sync_copy(lvl_hbm.at[wid], lvl_v)

    def b_step(b, carry):
        def q_step(j, c):
            v = x_v[pl.ds(b * _S + j * 16, 16)]
            q = v * jnp.float32(_L - 1) + jnp.float32(0.5)
            qi = jnp.clip(q.astype(jnp.int32), 0, _L - 1)
            idx_v[pl.ds(j * 16, 16)] = qi
            return c

        lax.fori_loop(0, _S // 16, q_step, 0, unroll=4)

        def s_step(t, fa):
            f0, f1 = fa
            base = t * 16
            ivec = idx_v[pl.ds(base, 16)] * _DT
            for j in range(16):
                po = (base + j) * _DT
                io = ivec[j]
                f0 = f0 + pos_v[pl.ds(po, 16)] * lvl_v[pl.ds(io, 16)]
                f1 = f1 + pos_v[pl.ds(po + 16, 16)] * lvl_v[pl.ds(io + 16, 16)]
            return (f0, f1)

        z = jnp.zeros((16,), jnp.float32)
        f0, f1 = lax.fori_loop(0, _S // 16, s_step, (z, z))

        ob = b * _DT
        out_v[pl.ds(ob, 16)] = jnp.where(f0 > 0, 1.0, -1.0).astype(jnp.float32)
        out_v[pl.ds(ob + 16, 16)] = jnp.where(f1 > 0, 1.0, -1.0).astype(jnp.float32)
        return carry

    lax.fori_loop(0, _B, b_step, 0)
    pltpu.sync_copy(out_v, out_hbm.at[wid])


@jax.jit
def kernel(x, position_weight, level_weight):
    pos_t = position_weight.reshape(_S, _NW, _DT).transpose(1, 0, 2)
    pos_t = pos_t.reshape(_NW, _S * _DT)
    lvl_t = level_weight.reshape(_L, _NW, _DT).transpose(1, 0, 2)
    lvl_t = lvl_t.reshape(_NW, _L * _DT)
    xf = x.reshape(_B * _S)

    mesh = plsc.VectorSubcoreMesh(core_axis_name="c", subcore_axis_name="s")
    run = functools.partial(
        pl.kernel,
        mesh=mesh,
        out_type=jax.ShapeDtypeStruct((_NW, _B * _DT), jnp.float32),
        scratch_types=[
            pltpu.VMEM((_B * _S,), jnp.float32),
            pltpu.VMEM((_S * _DT,), jnp.float32),
            pltpu.VMEM((_L * _DT,), jnp.float32),
            pltpu.VMEM((_S,), jnp.int32),
            pltpu.VMEM((_B * _DT,), jnp.float32),
        ],
    )(_sc_body)
    out = run(xf, pos_t, lvl_t)
    return out.reshape(_NW, _B, _DT).transpose(1, 0, 2).reshape(_B, _D)
